# baseline (speedup 1.0000x reference)
import functools

import jax
import jax.numpy as jnp
from jax import lax
from jax.experimental import pallas as pl
from jax.experimental.pallas import tpu as pltpu
from jax.experimental.pallas import tpu_sc as plsc

_NC = 2
_NS = 16
_L = 16
_NW = _NC * _NS
_CHUNK = 128


@functools.lru_cache(maxsize=None)
def _make_sc_kernel(batch: int, dim: int):
    b_w = batch // _NW
    n_chunks = b_w // _CHUNK
    n_groups = b_w // _L
    mesh = plsc.VectorSubcoreMesh(
        core_axis_name="c", subcore_axis_name="s",
        num_cores=_NC, num_subcores=_NS)

    @functools.partial(
        pl.kernel,
        out_type=jax.ShapeDtypeStruct((batch,), jnp.float32),
        mesh=mesh,
        compiler_params=pltpu.CompilerParams(
            needs_layout_passes=False, use_tc_tiling_on_sc=False),
        scratch_types=[
            pltpu.VMEM((b_w,), jnp.int32),
            pltpu.VMEM((b_w,), jnp.int32),
            pltpu.VMEM((b_w, dim), jnp.float32),
            pltpu.VMEM((b_w, dim), jnp.float32),
            pltpu.VMEM((b_w,), jnp.float32),
            pltpu.SemaphoreType.DMA,
        ],
    )
    def two_tower(uid_hbm, gid_hbm, ut_hbm, gt_hbm, out_hbm,
                  uidx, gidx, urows, grows, out_v, sem):
        wid = lax.axis_index("s") * _NC + lax.axis_index("c")
        base = wid * b_w
        pltpu.sync_copy(uid_hbm.at[pl.ds(base, b_w)], uidx)
        pltpu.sync_copy(gid_hbm.at[pl.ds(base, b_w)], gidx)

        copies = []
        for j in range(n_chunks):
            sl = pl.ds(j * _CHUNK, _CHUNK)
            copies.append(pltpu.async_copy(ut_hbm.at[uidx.at[sl]], urows.at[sl], sem))
            copies.append(pltpu.async_copy(gt_hbm.at[gidx.at[sl]], grows.at[sl], sem))
        for c in copies:
            c.wait()

        lane = lax.iota(jnp.int32, _L)

        def group(g, carry):
            rows = g * _L + lane
            acc = jnp.zeros((_L,), jnp.float32)
            for d in range(dim):
                cols = jnp.full((_L,), d, jnp.int32)
                u = plsc.load_gather(urows, [rows, cols])
                v = plsc.load_gather(grows, [rows, cols])
                acc = acc + u * v
            out_v[pl.ds(g * _L, _L)] = 1.0 / (1.0 + jnp.exp(-acc))
            return carry

        lax.fori_loop(0, n_groups, group, 0)
        pltpu.sync_copy(out_v, out_hbm.at[pl.ds(base, b_w)])

    return two_tower


@functools.lru_cache(maxsize=None)
def _make_tc_transpose(n: int, dim: int, blk: int):
    def body(i_ref, o_ref):
        eye = jnp.eye(dim, dtype=jnp.float32)
        o_ref[...] = jax.lax.dot_general(
            i_ref[...], eye, (((0,), (0,)), ((), ())),
            precision=jax.lax.Precision.HIGHEST)

    return pl.pallas_call(
        body,
        grid=(pl.cdiv(n, blk),),
        in_specs=[pl.BlockSpec((dim, blk), lambda j: (0, j))],
        out_specs=pl.BlockSpec((blk, dim), lambda j: (j, 0)),
        out_shape=jax.ShapeDtypeStruct((n, dim), jnp.float32),
    )


def kernel(user_ids, game_ids, user_table, game_table):
    fn = _make_sc_kernel(user_ids.shape[0], user_table.shape[1])
    nu, dim = user_table.shape
    u_lin = _make_tc_transpose(nu, dim, 8192)(user_table.T)
    return fn(user_ids.astype(jnp.int32), game_ids.astype(jnp.int32),
              u_lin, game_table)


# speedup vs baseline: 2.5944x; 2.5944x over previous
import functools

import jax
import jax.numpy as jnp
from jax import lax
from jax.experimental import pallas as pl
from jax.experimental.pallas import tpu as pltpu
from jax.experimental.pallas import tpu_sc as plsc

_NC = 2
_NS = 16
_L = 16
_NW = _NC * _NS
_CHUNK = 128
_HALF = 256
_BLK = 8192


def _pack_rows(n: int, dim: int):
    pack = 128 // dim
    p = 1 << ((n + pack - 1) // pack - 1).bit_length()
    return pack, p


@functools.lru_cache(maxsize=None)
def _make_sc_kernel(batch: int, dim: int, pu: int, pg: int):
    b_w = batch // _NW
    su = (pu - 1).bit_length()
    sg = (pg - 1).bit_length()
    mesh = plsc.VectorSubcoreMesh(
        core_axis_name="c", subcore_axis_name="s",
        num_cores=_NC, num_subcores=_NS)

    @functools.partial(
        pl.kernel,
        out_type=jax.ShapeDtypeStruct((batch,), jnp.float32),
        mesh=mesh,
        compiler_params=pltpu.CompilerParams(
            needs_layout_passes=False, use_tc_tiling_on_sc=False),
        scratch_types=[
            pltpu.VMEM((b_w,), jnp.int32),
            pltpu.VMEM((b_w,), jnp.int32),
            pltpu.VMEM((b_w,), jnp.int32),
            pltpu.VMEM((b_w,), jnp.int32),
            pltpu.VMEM((_HALF, 128), jnp.float32),
            pltpu.VMEM((_HALF, 128), jnp.float32),
            pltpu.VMEM((b_w,), jnp.float32),
            pltpu.SemaphoreType.DMA,
        ],
    )
    def two_tower(uid_hbm, gid_hbm, ut4_hbm, gt4_hbm, out_hbm,
                  uidx, gidx, urow, grow, ubuf, gbuf, out_v, sem):
        wid = lax.axis_index("s") * _NC + lax.axis_index("c")
        base = wid * b_w
        pltpu.sync_copy(uid_hbm.at[pl.ds(base, b_w)], uidx)
        pltpu.sync_copy(gid_hbm.at[pl.ds(base, b_w)], gidx)

        def rows(v, carry):
            sl = pl.ds(v * _L, _L)
            urow[sl] = uidx[sl] & (pu - 1)
            grow[sl] = gidx[sl] & (pg - 1)
            return carry

        lax.fori_loop(0, b_w // _L, rows, 0)

        lane = lax.iota(jnp.int32, _L)

        def half(h, carry):
            hbase = h * _HALF
            copies = []
            for j in range(_HALF // _CHUNK):
                isl = pl.ds(hbase + j * _CHUNK, _CHUNK)
                osl = pl.ds(j * _CHUNK, _CHUNK)
                copies.append(pltpu.async_copy(
                    ut4_hbm.at[urow.at[isl]], ubuf.at[osl], sem))
                copies.append(pltpu.async_copy(
                    gt4_hbm.at[grow.at[isl]], gbuf.at[osl], sem))
            for c in copies:
                c.wait()

            def group(g, carry2):
                sl = pl.ds(hbase + g * _L, _L)
                ucol = (uidx[sl] >> su) * dim
                gcol = (gidx[sl] >> sg) * dim
                r16 = g * _L + lane
                acc = jnp.zeros((_L,), jnp.float32)
                for d in range(dim):
                    u = plsc.load_gather(ubuf, [r16, ucol + d])
                    v = plsc.load_gather(gbuf, [r16, gcol + d])
                    acc = acc + u * v
                out_v[sl] = 1.0 / (1.0 + jnp.exp(-acc))
                return carry2

            lax.fori_loop(0, _HALF // _L, group, 0)
            return carry

        lax.fori_loop(0, b_w // _HALF, half, 0)
        pltpu.sync_copy(out_v, out_hbm.at[pl.ds(base, b_w)])

    return two_tower


@functools.lru_cache(maxsize=None)
def _make_tc_pack(n: int, dim: int):
    pack, p = _pack_rows(n, dim)
    assert p % _BLK == 0
    last = pl.cdiv(n, _BLK) - 1

    def body(*refs):
        in_refs, o_ref = refs[:pack], refs[pack]
        for m in range(pack):
            o_ref[:, m * dim:(m + 1) * dim] = in_refs[m][...].T

    def imap(mm, j):
        return (0, jnp.minimum(j + mm * (p // _BLK), last))

    return pl.pallas_call(
        body,
        grid=(p // _BLK,),
        in_specs=[
            pl.BlockSpec((dim, _BLK), functools.partial(imap, m))
            for m in range(pack)
        ],
        out_specs=pl.BlockSpec((_BLK, 128), lambda j: (j, 0)),
        out_shape=jax.ShapeDtypeStruct((p, 128), jnp.float32),
    )


def kernel(user_ids, game_ids, user_table, game_table):
    nu, dim = user_table.shape
    ng = game_table.shape[0]
    _, pu = _pack_rows(nu, dim)
    _, pg = _pack_rows(ng, dim)
    fn = _make_sc_kernel(user_ids.shape[0], dim, pu, pg)
    utT = user_table.T
    gtT = game_table.T
    ut4 = _make_tc_pack(nu, dim)(*([utT] * (128 // dim)))
    gt4 = _make_tc_pack(ng, dim)(*([gtT] * (128 // dim)))
    return fn(user_ids.astype(jnp.int32), game_ids.astype(jnp.int32),
              ut4, gt4)
